# baseline (speedup 1.0000x reference)
import jax
import jax.numpy as jnp
from jax.experimental import pallas as pl
from jax.experimental.pallas import tpu as pltpu

B, IN_DIM, PROJ_DIM, K = 8192, 1408, 256, 64
COMMIT = 0.25
BM = 1024
NB = B // BM


def _body(z_ref, w_ref, b_ref, g_ref, be_ref, cb_ref, zq_ref, idx_ref, loss_ref):
    z = z_ref[...]
    nrm = jnp.sqrt(jnp.sum(z * z, axis=1, keepdims=True))
    zn = z / jnp.maximum(nrm, 1e-12)
    zl = jax.lax.dot_general(
        zn, w_ref[...], (((1,), (1,)), ((), ())),
        preferred_element_type=jnp.float32) + b_ref[...]
    mu = jnp.mean(zl, axis=1, keepdims=True)
    d = zl - mu
    var = jnp.mean(d * d, axis=1, keepdims=True)
    zp = d / jnp.sqrt(var + 1e-5) * g_ref[...] + be_ref[...]
    pn2 = jnp.sum(zp * zp, axis=1, keepdims=True)
    zf = zp / jnp.maximum(jnp.sqrt(pn2), 1e-12)
    cb = cb_ref[...]
    scores = jax.lax.dot_general(
        zf, cb, (((1,), (1,)), ((), ())),
        preferred_element_type=jnp.float32)
    zsq = jnp.sum(zf * zf, axis=1, keepdims=True)
    cbsq = jnp.sum(cb * cb, axis=1)
    dist = zsq - 2.0 * scores + cbsq[None, :]
    idx = jnp.argmin(dist, axis=1).astype(jnp.int32)
    onehot = (jax.lax.broadcasted_iota(jnp.int32, (BM, K), 1)
              == idx[:, None]).astype(jnp.float32)
    zq = jax.lax.dot_general(
        onehot, cb, (((1,), (0,)), ((), ())),
        preferred_element_type=jnp.float32)
    zq_ref[...] = zf + (zq - zf)
    idx_ref[0, 0, :] = idx
    loss_ref[...] = jnp.sum((zq - zf) ** 2).reshape(1, 1, 1)


@jax.jit
def kernel(z_frame, W, b, gamma, beta, codebook):
    b2 = b.reshape(1, PROJ_DIM)
    g2 = gamma.reshape(1, PROJ_DIM)
    be2 = beta.reshape(1, PROJ_DIM)
    zq, idx3, loss_acc = pl.pallas_call(
        _body,
        grid=(NB,),
        in_specs=[
            pl.BlockSpec((BM, IN_DIM), lambda i: (i, 0)),
            pl.BlockSpec((PROJ_DIM, IN_DIM), lambda i: (0, 0)),
            pl.BlockSpec((1, PROJ_DIM), lambda i: (0, 0)),
            pl.BlockSpec((1, PROJ_DIM), lambda i: (0, 0)),
            pl.BlockSpec((1, PROJ_DIM), lambda i: (0, 0)),
            pl.BlockSpec((K, PROJ_DIM), lambda i: (0, 0)),
        ],
        out_specs=[
            pl.BlockSpec((BM, PROJ_DIM), lambda i: (i, 0)),
            pl.BlockSpec((1, 1, BM), lambda i: (i, 0, 0)),
            pl.BlockSpec((1, 1, 1), lambda i: (i, 0, 0)),
        ],
        out_shape=[
            jax.ShapeDtypeStruct((B, PROJ_DIM), jnp.float32),
            jax.ShapeDtypeStruct((NB, 1, BM), jnp.int32),
            jax.ShapeDtypeStruct((NB, 1, 1), jnp.float32),
        ],
        compiler_params=pltpu.CompilerParams(
            dimension_semantics=("arbitrary",),
        ),
    )(z_frame, W, b2, g2, be2, codebook)
    indices = idx3.reshape(B)
    loss = jnp.sum(loss_acc) * (COMMIT / (B * PROJ_DIM))
    return zq, indices, loss


# speedup vs baseline: 1.0460x; 1.0460x over previous
import jax
import jax.numpy as jnp
from jax.experimental import pallas as pl
from jax.experimental.pallas import tpu as pltpu

B, IN_DIM, PROJ_DIM, K = 8192, 1408, 256, 64
COMMIT = 0.25
BM = 1024
NB = B // BM


def _body(z_ref, w_ref, b_ref, g_ref, be_ref, cb_ref, zq_ref, idx_ref, loss_ref):
    z = z_ref[...]
    nrm = jnp.sqrt(jnp.sum(z * z, axis=1, keepdims=True))
    zn = z / jnp.maximum(nrm, 1e-12)
    zl = jax.lax.dot_general(
        zn, w_ref[...], (((1,), (1,)), ((), ())),
        preferred_element_type=jnp.float32) + b_ref[...]
    mu = jnp.mean(zl, axis=1, keepdims=True)
    d = zl - mu
    var = jnp.mean(d * d, axis=1, keepdims=True)
    zp = d / jnp.sqrt(var + 1e-5) * g_ref[...] + be_ref[...]
    pn2 = jnp.sum(zp * zp, axis=1, keepdims=True)
    zf = zp / jnp.maximum(jnp.sqrt(pn2), 1e-12)
    cb = cb_ref[...]
    scores = jax.lax.dot_general(
        zf, cb, (((1,), (1,)), ((), ())),
        preferred_element_type=jnp.float32)
    zsq = jnp.sum(zf * zf, axis=1, keepdims=True)
    cbsq = jnp.sum(cb * cb, axis=1)
    dist = zsq - 2.0 * scores + cbsq[None, :]
    idx = jnp.argmin(dist, axis=1).astype(jnp.int32)
    onehot = (jax.lax.broadcasted_iota(jnp.int32, (BM, K), 1)
              == idx[:, None]).astype(jnp.float32)
    zq = jax.lax.dot_general(
        onehot, cb, (((1,), (0,)), ((), ())),
        preferred_element_type=jnp.float32)
    zq_ref[...] = zf + (zq - zf)
    idx_ref[0, 0, :] = idx
    part = jnp.sum((zq - zf) ** 2).reshape(1, 1)

    @pl.when(pl.program_id(0) == 0)
    def _init():
        loss_ref[...] = jnp.zeros((1, 1), jnp.float32)

    loss_ref[...] += part


@jax.jit
def kernel(z_frame, W, b, gamma, beta, codebook):
    b2 = b.reshape(1, PROJ_DIM)
    g2 = gamma.reshape(1, PROJ_DIM)
    be2 = beta.reshape(1, PROJ_DIM)
    zq, idx3, loss_acc = pl.pallas_call(
        _body,
        grid=(NB,),
        in_specs=[
            pl.BlockSpec((BM, IN_DIM), lambda i: (i, 0)),
            pl.BlockSpec((PROJ_DIM, IN_DIM), lambda i: (0, 0)),
            pl.BlockSpec((1, PROJ_DIM), lambda i: (0, 0)),
            pl.BlockSpec((1, PROJ_DIM), lambda i: (0, 0)),
            pl.BlockSpec((1, PROJ_DIM), lambda i: (0, 0)),
            pl.BlockSpec((K, PROJ_DIM), lambda i: (0, 0)),
        ],
        out_specs=[
            pl.BlockSpec((BM, PROJ_DIM), lambda i: (i, 0)),
            pl.BlockSpec((1, 1, BM), lambda i: (i, 0, 0)),
            pl.BlockSpec((1, 1), lambda i: (0, 0)),
        ],
        out_shape=[
            jax.ShapeDtypeStruct((B, PROJ_DIM), jnp.float32),
            jax.ShapeDtypeStruct((NB, 1, BM), jnp.int32),
            jax.ShapeDtypeStruct((1, 1), jnp.float32),
        ],
        compiler_params=pltpu.CompilerParams(
            dimension_semantics=("arbitrary",),
        ),
    )(z_frame, W, b2, g2, be2, codebook)
    indices = idx3.reshape(B)
    loss = loss_acc[0, 0] * (COMMIT / (B * PROJ_DIM))
    return zq, indices, loss
